# Initial kernel scaffold; baseline (speedup 1.0000x reference)
import functools

import jax
import jax.numpy as jnp
from jax import lax
from jax.experimental import pallas as pl
from jax.experimental.pallas import tpu as pltpu
from jax.experimental.pallas import tpu_sc as plsc

BATCH = 4096
FIELDS = 26
DIM = 64
TOTAL = BATCH * FIELDS
NC = 2
NS = 16
NW = NC * NS
PER_W = TOTAL // NW
CH = 128
NCH = PER_W // CH

_mesh = plsc.VectorSubcoreMesh(core_axis_name="c", subcore_axis_name="s")


@functools.partial(
    pl.kernel,
    mesh=_mesh,
    out_type=jax.ShapeDtypeStruct((TOTAL, DIM), jnp.float32),
    scratch_types=[
        pltpu.VMEM((NCH, CH), jnp.int32),
        pltpu.VMEM((CH, DIM), jnp.float32),
        pltpu.SemaphoreType.DMA,
    ],
)
def _gather_kernel(idx_hbm, table_hbm, out_hbm, idx_v, buf, gsem):
    wid = lax.axis_index("s") * NC + lax.axis_index("c")
    base = wid * PER_W
    pltpu.sync_copy(idx_hbm.at[wid], idx_v)

    def body(j, carry):
        pltpu.async_copy(table_hbm.at[idx_v.at[j]], buf, gsem).wait()
        pltpu.sync_copy(buf, out_hbm.at[pl.ds(base + j * CH, CH)])
        return carry

    lax.fori_loop(0, NCH, body, 0)


def kernel(input, embeddings):
    idx = input.reshape(NW, NCH, CH)
    out = _gather_kernel(idx, embeddings)
    return out.reshape(BATCH, FIELDS, DIM)


# speedup vs baseline: 1.1011x; 1.1011x over previous
import functools

import jax
import jax.numpy as jnp
from jax import lax
from jax.experimental import pallas as pl
from jax.experimental.pallas import tpu as pltpu
from jax.experimental.pallas import tpu_sc as plsc

BATCH = 4096
FIELDS = 26
DIM = 64
TOTAL = BATCH * FIELDS
NC = 2
NS = 16
NW = NC * NS
PER_W = TOTAL // NW
CH = 128
NCH = PER_W // CH

_mesh = plsc.VectorSubcoreMesh(core_axis_name="c", subcore_axis_name="s")


@functools.partial(
    pl.kernel,
    mesh=_mesh,
    out_type=jax.ShapeDtypeStruct((TOTAL, DIM), jnp.float32),
    scratch_types=[
        pltpu.VMEM((NCH, CH), jnp.int32),
        pltpu.VMEM((CH, DIM), jnp.float32),
        pltpu.SemaphoreType.DMA,
    ],
    compiler_params=pltpu.CompilerParams(use_tc_tiling_on_sc=False),
)
def _gather_kernel(idx_hbm, table_hbm, out_hbm, idx_v, buf, gsem):
    wid = lax.axis_index("s") * NC + lax.axis_index("c")
    base = wid * PER_W
    pltpu.sync_copy(idx_hbm.at[wid], idx_v)

    def body(j, carry):
        pltpu.async_copy(table_hbm.at[idx_v.at[j]], buf, gsem).wait()
        pltpu.sync_copy(buf, out_hbm.at[pl.ds(base + j * CH, CH)])
        return carry

    lax.fori_loop(0, NCH, body, 0)


def kernel(input, embeddings):
    idx = input.reshape(NW, NCH, CH)
    out = _gather_kernel(idx, embeddings)
    return out.reshape(BATCH, FIELDS, DIM)


# speedup vs baseline: 1.2091x; 1.0981x over previous
import functools

import jax
import jax.numpy as jnp
from jax import lax
from jax.experimental import pallas as pl
from jax.experimental.pallas import tpu as pltpu
from jax.experimental.pallas import tpu_sc as plsc

BATCH = 4096
FIELDS = 26
DIM = 64
TOTAL = BATCH * FIELDS
NC = 2
NS = 16
NW = NC * NS
PER_W = TOTAL // NW
CH = 104
NCH = PER_W // CH
NSLOT = 8
NGRP = NCH // 4

_mesh = plsc.VectorSubcoreMesh(core_axis_name="c", subcore_axis_name="s")


@functools.partial(
    pl.kernel,
    mesh=_mesh,
    out_type=jax.ShapeDtypeStruct((TOTAL, DIM), jnp.float32),
    scratch_types=(
        [pltpu.VMEM((NCH, CH), jnp.int32)]
        + [pltpu.VMEM((CH, DIM), jnp.float32) for _ in range(NSLOT)]
        + [pltpu.SemaphoreType.DMA for _ in range(2 * NSLOT)]
    ),
    compiler_params=pltpu.CompilerParams(use_tc_tiling_on_sc=False),
)
def _gather_kernel(idx_hbm, table_hbm, out_hbm, idx_v, *scratch):
    bufs = scratch[:NSLOT]
    gsem = scratch[NSLOT:2 * NSLOT]
    osem = scratch[2 * NSLOT:]
    wid = lax.axis_index("s") * NC + lax.axis_index("c")
    base = wid * PER_W
    pltpu.sync_copy(idx_hbm.at[wid], idx_v)

    def fire_gather(k, s):
        pltpu.async_copy(table_hbm.at[idx_v.at[k]], bufs[s], gsem[s])

    def wait_gather(k, s):
        pltpu.make_async_copy(table_hbm.at[idx_v.at[k]], bufs[s], gsem[s]).wait()

    def fire_out(k, s):
        pltpu.async_copy(bufs[s], out_hbm.at[pl.ds(base + k * CH, CH)], osem[s])

    def wait_out(k, s):
        pltpu.make_async_copy(
            bufs[s], out_hbm.at[pl.ds(base + k * CH, CH)], osem[s]).wait()

    def handle_group(g, slot0, wait_o):
        for b in range(4):
            k = g * 4 + b
            s = slot0 + b
            wait_gather(k, s)
            fire_out(k, s)
            s2 = (slot0 + 4) % NSLOT + b
            if wait_o:
                wait_out(k - 4, s2)
            fire_gather(k + 4, s2)

    for b in range(4):
        fire_gather(b, b)
    handle_group(0, 0, wait_o=False)

    def body(j2, carry):
        g = 1 + 2 * j2
        handle_group(g, 4, wait_o=True)
        handle_group(g + 1, 0, wait_o=True)
        return carry

    lax.fori_loop(0, (NGRP - 2) // 2, body, 0)

    for b in range(4):
        k = (NGRP - 1) * 4 + b
        wait_gather(k, 4 + b)
        fire_out(k, 4 + b)
    for b in range(4):
        wait_out((NGRP - 2) * 4 + b, b)
        wait_out((NGRP - 1) * 4 + b, 4 + b)


def kernel(input, embeddings):
    idx = input.reshape(NW, NCH, CH)
    out = _gather_kernel(idx, embeddings)
    return out.reshape(BATCH, FIELDS, DIM)


# speedup vs baseline: 1.2140x; 1.0040x over previous
import functools

import jax
import jax.numpy as jnp
from jax import lax
from jax.experimental import pallas as pl
from jax.experimental.pallas import tpu as pltpu
from jax.experimental.pallas import tpu_sc as plsc

BATCH = 4096
FIELDS = 26
DIM = 64
NTAB = 100000
NC = 2
NS = 16
NW = NC * NS
DPW = DIM // NW

_mesh = plsc.VectorSubcoreMesh(core_axis_name="c", subcore_axis_name="s")


@functools.partial(
    pl.kernel,
    mesh=_mesh,
    out_type=jax.ShapeDtypeStruct((FIELDS, DIM, BATCH), jnp.float32),
    scratch_types=[
        pltpu.VMEM((NTAB,), jnp.float32),
        pltpu.VMEM((BATCH,), jnp.int32),
        pltpu.VMEM((BATCH,), jnp.float32),
    ],
    compiler_params=pltpu.CompilerParams(
        use_tc_tiling_on_sc=True, needs_layout_passes=False),
)
def _gather_kernel(idx_hbm, tab_hbm, out_hbm, row_v, idx_v, out_v):
    wid = lax.axis_index("s") * NC + lax.axis_index("c")

    def d_body(dd, carry):
        d = dd * NW + wid
        pltpu.sync_copy(tab_hbm.at[d], row_v)

        def f_body(f, carry2):
            pltpu.sync_copy(idx_hbm.at[f], idx_v)

            def g_body(i, carry3):
                ids = idx_v[pl.ds(i * 16, 16)]
                out_v[pl.ds(i * 16, 16)] = plsc.load_gather(row_v, [ids])
                return carry3

            lax.fori_loop(0, BATCH // 16, g_body, 0)
            pltpu.sync_copy(out_v, out_hbm.at[f, d])
            return carry2

        lax.fori_loop(0, FIELDS, f_body, 0)
        return carry

    lax.fori_loop(0, DPW, d_body, 0)


def kernel(input, embeddings):
    out_t = _gather_kernel(input.T, embeddings.T)
    return jnp.transpose(out_t, (2, 0, 1))


# speedup vs baseline: 2.3428x; 1.9299x over previous
import functools

import jax
import jax.numpy as jnp
from jax import lax
from jax.experimental import pallas as pl
from jax.experimental.pallas import tpu as pltpu
from jax.experimental.pallas import tpu_sc as plsc

BATCH = 4096
FIELDS = 26
DIM = 64
NTAB = 100000
NC = 2
NS = 16
NW = NC * NS
DPW = DIM // NW
NVEC = BATCH // 16

_mesh = plsc.VectorSubcoreMesh(core_axis_name="c", subcore_axis_name="s")


@functools.partial(
    pl.kernel,
    mesh=_mesh,
    out_type=jax.ShapeDtypeStruct((FIELDS, DIM, BATCH), jnp.float32),
    scratch_types=[
        pltpu.VMEM((NTAB,), jnp.float32),
        pltpu.VMEM((BATCH,), jnp.int32),
        pltpu.VMEM((BATCH,), jnp.int32),
        pltpu.VMEM((BATCH,), jnp.float32),
        pltpu.VMEM((BATCH,), jnp.float32),
        pltpu.SemaphoreType.DMA,
        pltpu.SemaphoreType.DMA,
        pltpu.SemaphoreType.DMA,
        pltpu.SemaphoreType.DMA,
    ],
    compiler_params=pltpu.CompilerParams(
        use_tc_tiling_on_sc=True, needs_layout_passes=False),
)
def _gather_kernel(idx_hbm, tab_hbm, out_hbm, row_v, ib0, ib1, ob0, ob1,
                   is0, is1, os0, os1):
    ibufs, isems = (ib0, ib1), (is0, is1)
    obufs, osems = (ob0, ob1), (os0, os1)
    wid = lax.axis_index("s") * NC + lax.axis_index("c")

    def fire_idx(f, b):
        pltpu.async_copy(idx_hbm.at[f], ibufs[b], isems[b])

    def wait_idx(f, b):
        pltpu.make_async_copy(idx_hbm.at[f], ibufs[b], isems[b]).wait()

    def fire_out(f, d, b):
        pltpu.async_copy(obufs[b], out_hbm.at[f, d], osems[b])

    def wait_out(f, d, b):
        pltpu.make_async_copy(obufs[b], out_hbm.at[f, d], osems[b]).wait()

    def gather(b):
        @plsc.parallel_loop(0, NVEC, unroll=8)
        def _(i):
            ids = ibufs[b][pl.ds(i * 16, 16)]
            obufs[b][pl.ds(i * 16, 16)] = plsc.load_gather(row_v, [ids])

    for dd in range(DPW):
        d = dd * NW + wid
        pltpu.sync_copy(tab_hbm.at[d], row_v)
        fire_idx(0, 0)

        def jbody(j, carry):
            for b in range(2):
                f = 2 * j + b
                wait_idx(f, b)

                @pl.when(f + 1 < FIELDS)
                def _():
                    fire_idx(f + 1, 1 - b)

                @pl.when(j >= 1)
                def _():
                    wait_out(f - 2, d, b)

                gather(b)
                fire_out(f, d, b)
            return carry

        lax.fori_loop(0, FIELDS // 2, jbody, 0)
        wait_out(FIELDS - 2, d, 0)
        wait_out(FIELDS - 1, d, 1)


def kernel(input, embeddings):
    out_t = _gather_kernel(input.T, embeddings.T)
    return jnp.transpose(out_t, (2, 0, 1))


# speedup vs baseline: 3.0088x; 1.2843x over previous
import functools

import jax
import jax.numpy as jnp
from jax import lax
from jax.experimental import pallas as pl
from jax.experimental.pallas import tpu as pltpu
from jax.experimental.pallas import tpu_sc as plsc

BATCH = 4096
FIELDS = 26
DIM = 64
NTAB = 100000
NC = 2
NS = 16
NW = NC * NS
DPW = DIM // NW
NVEC = BATCH // 16
IR = 4
OR = 3
UNROLL = 12
NMAIN = (FIELDS // UNROLL) * UNROLL

_mesh = plsc.VectorSubcoreMesh(core_axis_name="c", subcore_axis_name="s")


@functools.partial(
    pl.kernel,
    mesh=_mesh,
    out_type=jax.ShapeDtypeStruct((FIELDS, DIM, BATCH), jnp.float32),
    scratch_types=(
        [pltpu.VMEM((NTAB,), jnp.float32)]
        + [pltpu.VMEM((BATCH,), jnp.int32) for _ in range(IR)]
        + [pltpu.VMEM((BATCH,), jnp.float32) for _ in range(OR)]
        + [pltpu.SemaphoreType.DMA for _ in range(IR + OR)]
    ),
    compiler_params=pltpu.CompilerParams(
        use_tc_tiling_on_sc=True, needs_layout_passes=False),
)
def _gather_kernel(idx_hbm, tab_hbm, out_hbm, row_v, *scr):
    ibufs = scr[:IR]
    obufs = scr[IR:IR + OR]
    isems = scr[IR + OR:2 * IR + OR]
    osems = scr[2 * IR + OR:]
    wid = lax.axis_index("s") * NC + lax.axis_index("c")

    def fire_idx(f, b):
        pltpu.async_copy(idx_hbm.at[f], ibufs[b], isems[b])

    def wait_idx(f, b):
        pltpu.make_async_copy(idx_hbm.at[f], ibufs[b], isems[b]).wait()

    def fire_out(f, d, b):
        pltpu.async_copy(obufs[b], out_hbm.at[f, d], osems[b])

    def wait_out(f, d, b):
        pltpu.make_async_copy(obufs[b], out_hbm.at[f, d], osems[b]).wait()

    def gather(bi, bo):
        @plsc.parallel_loop(0, NVEC, unroll=8)
        def _(i):
            ids = ibufs[bi][pl.ds(i * 16, 16)]
            obufs[bo][pl.ds(i * 16, 16)] = plsc.load_gather(row_v, [ids])

    def d_body(dd, carry):
        d = dd * NW + wid
        pltpu.sync_copy(tab_hbm.at[d], row_v)
        for p in range(IR - 1):
            fire_idx(p, p)

        def step(f, m):
            wait_idx(f, m % IR)

            @pl.when(f + IR - 1 < FIELDS)
            def _():
                fire_idx(f + IR - 1, (m + IR - 1) % IR)

            @pl.when(f >= OR)
            def _():
                wait_out(f - OR, d, m % OR)

            gather(m % IR, m % OR)
            fire_out(f, d, m % OR)

        def jbody(j, carry2):
            for m in range(UNROLL):
                step(j * UNROLL + m, m)
            return carry2

        lax.fori_loop(0, NMAIN // UNROLL, jbody, 0)
        for m in range(NMAIN, FIELDS):
            step(m, m)
        for f in range(FIELDS - OR, FIELDS):
            wait_out(f, d, f % OR)
        return carry

    lax.fori_loop(0, DPW, d_body, 0)


def kernel(input, embeddings):
    out_t = _gather_kernel(input.T, embeddings.T)
    return jnp.transpose(out_t, (2, 0, 1))
